# baseline (speedup 1.0000x reference)
import jax
import jax.numpy as jnp
from jax.experimental import pallas as pl
from jax.experimental.pallas import tpu as pltpu

B = 512
N = 64
D = 512
O = 128
BB = 128


def _fused_kernel(states_ref, adj_ref, wg_ref, bg_ref, wc_ref,
                  bc_ref, outs_ref, vals_ref):
    st = states_ref[...]
    adjf = adj_ref[...]

    wg = wg_ref[...].astype(jnp.bfloat16)
    msgs = [
        jax.lax.dot_general(
            st[:, n * D:(n + 1) * D].astype(jnp.bfloat16), wg,
            (((1,), (0,)), ((), ())),
            preferred_element_type=jnp.float32).astype(jnp.bfloat16)
        for n in range(N)
    ]
    msg = jnp.swapaxes(jnp.stack(msgs, axis=0), 0, 1)

    adj = adjf.reshape(BB, N, N)
    eye = (jax.lax.broadcasted_iota(jnp.int32, (N, N), 0) ==
           jax.lax.broadcasted_iota(jnp.int32, (N, N), 1)).astype(jnp.float32)
    a_hat = adj + eye[None, :, :]
    deg = jnp.sum(a_hat, axis=1)
    dinv = jnp.where(deg > 0, jax.lax.rsqrt(deg), 0.0)
    aw = a_hat * dinv[:, :, None] * dinv[:, None, :]

    out = jax.lax.dot_general(
        aw.astype(jnp.bfloat16), msg,
        (((1,), (1,)), ((0,), (0,))),
        preferred_element_type=jnp.float32)
    outs_ref[...] = (out + bg_ref[...][None, None, :]).reshape(BB, N * O)

    wc = wc_ref[...]
    v_s = jnp.sum(st * wc[:, : N * D], axis=1)
    v_a = jnp.sum(adjf * wc[:, N * D:], axis=1)
    vals_ref[...] = (v_s + v_a + bc_ref[0, 0])[:, None]


def kernel(obs, W_gnn, b_gnn, W_crit, b_crit):
    wc = W_crit.reshape(1, -1)
    bc = b_crit.reshape(1, 1)

    grid = (B // BB,)
    outs, values = pl.pallas_call(
        _fused_kernel,
        grid=grid,
        in_specs=[
            pl.BlockSpec((BB, N * D), lambda i: (i, 0)),
            pl.BlockSpec((BB, N * N), lambda i: (i, N * D // (N * N))),
            pl.BlockSpec((D, O), lambda i: (0, 0)),
            pl.BlockSpec((O,), lambda i: (0,)),
            pl.BlockSpec((1, N * D + N * N), lambda i: (0, 0)),
            pl.BlockSpec((1, 1), lambda i: (0, 0)),
        ],
        out_specs=[
            pl.BlockSpec((BB, N * O), lambda i: (i, 0)),
            pl.BlockSpec((BB, 1), lambda i: (i, 0)),
        ],
        out_shape=[
            jax.ShapeDtypeStruct((B, N * O), jnp.float32),
            jax.ShapeDtypeStruct((B, 1), jnp.float32),
        ],
        compiler_params=pltpu.CompilerParams(
            dimension_semantics=("parallel",)),
    )(obs, obs, W_gnn, b_gnn, wc, bc)
    return outs, values


# speedup vs baseline: 1.0634x; 1.0634x over previous
import jax
import jax.numpy as jnp
from jax.experimental import pallas as pl

B = 512
N = 64
D = 512
O = 128
BB = 128


def _fused_kernel(states_ref, adj_ref, wg_ref, bg_ref, wcs_ref, wca_ref,
                  bc_ref, outs_ref, vals_ref):
    st = states_ref[...]
    adjf = adj_ref[...]

    wg = wg_ref[...].astype(jnp.bfloat16)
    msgs = [
        jax.lax.dot_general(
            st[:, n * D:(n + 1) * D].astype(jnp.bfloat16), wg,
            (((1,), (0,)), ((), ())),
            preferred_element_type=jnp.float32).astype(jnp.bfloat16)
        for n in range(N)
    ]
    msg = jnp.swapaxes(jnp.stack(msgs, axis=0), 0, 1)

    adj = adjf.reshape(BB, N, N)
    eye = (jax.lax.broadcasted_iota(jnp.int32, (N, N), 0) ==
           jax.lax.broadcasted_iota(jnp.int32, (N, N), 1)).astype(jnp.float32)
    a_hat = adj + eye[None, :, :]
    deg = jnp.sum(a_hat, axis=1)
    dinv = jnp.where(deg > 0, jax.lax.rsqrt(deg), 0.0)
    aw = a_hat * dinv[:, :, None] * dinv[:, None, :]

    out = jax.lax.dot_general(
        aw.astype(jnp.bfloat16), msg,
        (((1,), (1,)), ((0,), (0,))),
        preferred_element_type=jnp.float32)
    outs_ref[...] = (out + bg_ref[...][None, None, :]).reshape(BB, N * O)

    v_s = jnp.sum(st * wcs_ref[...], axis=1)
    v_a = jnp.sum(adjf * wca_ref[...], axis=1)
    vals_ref[...] = (v_s + v_a + bc_ref[0, 0])[:, None]


def kernel(obs, W_gnn, b_gnn, W_crit, b_crit):
    wc = W_crit.reshape(1, -1)
    wcs = wc[:, : N * D]
    wca = wc[:, N * D:]
    bc = b_crit.reshape(1, 1)

    grid = (B // BB,)
    outs, values = pl.pallas_call(
        _fused_kernel,
        grid=grid,
        in_specs=[
            pl.BlockSpec((BB, N * D), lambda i: (i, 0)),
            pl.BlockSpec((BB, N * N), lambda i: (i, N * D // (N * N))),
            pl.BlockSpec((D, O), lambda i: (0, 0)),
            pl.BlockSpec((O,), lambda i: (0,)),
            pl.BlockSpec((1, N * D), lambda i: (0, 0)),
            pl.BlockSpec((1, N * N), lambda i: (0, 0)),
            pl.BlockSpec((1, 1), lambda i: (0, 0)),
        ],
        out_specs=[
            pl.BlockSpec((BB, N * O), lambda i: (i, 0)),
            pl.BlockSpec((BB, 1), lambda i: (i, 0)),
        ],
        out_shape=[
            jax.ShapeDtypeStruct((B, N * O), jnp.float32),
            jax.ShapeDtypeStruct((B, 1), jnp.float32),
        ],
    )(obs, obs, W_gnn, b_gnn, wcs, wca, bc)
    return outs, values
